# baseline (speedup 1.0000x reference)
import functools

import jax
import jax.numpy as jnp
from jax import lax
from jax.experimental import pallas as pl
from jax.experimental.pallas import tpu as pltpu
from jax.experimental.pallas import tpu_sc as plsc

_D = 128
_CHUNK = 128
_NBUF = 6
_LAG = 3


@functools.partial(jax.jit, static_argnames=("b", "per_w", "nch"))
def _unpool(table, idx, *, b, per_w, nch):
    info = plsc.get_sparse_core_info()
    nc = info.num_cores

    mesh = plsc.VectorSubcoreMesh(core_axis_name="c", subcore_axis_name="s")

    @functools.partial(
        pl.kernel,
        mesh=mesh,
        out_type=jax.ShapeDtypeStruct((b, _D), jnp.float32),
        scratch_types=[
            pltpu.VMEM((per_w,), jnp.int32),
            pltpu.VMEM((_NBUF, _CHUNK, _D), jnp.float32),
            pltpu.SemaphoreType.DMA((_NBUF,)),
            pltpu.SemaphoreType.DMA((_NBUF,)),
        ],
    )
    def k(table_hbm, idx_hbm, out_hbm, idx_v, bufs, gsem, osem):
        wid = lax.axis_index("s") * nc + lax.axis_index("c")
        base = jnp.minimum(wid * per_w, b - per_w)
        pltpu.sync_copy(idx_hbm.at[pl.ds(base, per_w)], idx_v)

        offs = [min(j * _CHUNK, per_w - _CHUNK) for j in range(nch)]

        def start_gather(j):
            bf = j % _NBUF
            return pltpu.async_copy(
                table_hbm.at[idx_v.at[pl.ds(offs[j], _CHUNK)]],
                bufs.at[bf],
                gsem.at[bf],
            )

        go = [None] * nch
        oo = [None] * nch
        for j in range(min(_LAG, nch)):
            go[j] = start_gather(j)
        for j in range(nch):
            bf = j % _NBUF
            go[j].wait()
            oo[j] = pltpu.async_copy(
                bufs.at[bf],
                out_hbm.at[pl.ds(base + offs[j], _CHUNK)],
                osem.at[bf],
            )
            t = j + _LAG
            if t < nch:
                if t >= _NBUF:
                    oo[t - _NBUF].wait()
                go[t] = start_gather(t)
        for j in range(max(0, nch - _NBUF), nch):
            oo[j].wait()

    return k(table, idx)


def kernel(graph_feat, batch):
    b = batch.shape[0]
    info = plsc.get_sparse_core_info()
    nw = info.num_cores * info.num_subcores
    per_w = -(-b // nw)
    per_w = -(-per_w // 8) * 8
    per_w = max(per_w, _CHUNK)
    nch = -(-per_w // _CHUNK)
    return _unpool(graph_feat, batch.astype(jnp.int32), b=b, per_w=per_w, nch=nch)


# speedup vs baseline: 1.0190x; 1.0190x over previous
import functools

import jax
import jax.numpy as jnp
from jax import lax
from jax.experimental import pallas as pl
from jax.experimental.pallas import tpu as pltpu
from jax.experimental.pallas import tpu_sc as plsc

_D = 128
_G = 256
_CHUNK = 128
_NBUF = 3
_L = 16


@functools.partial(jax.jit, static_argnames=("b", "per_w", "nch"))
def _unpool(table, starts, meta, *, b, per_w, nch):
    info = plsc.get_sparse_core_info()
    nc = info.num_cores
    nmeta = meta.shape[0]

    mesh = plsc.VectorSubcoreMesh(core_axis_name="c", subcore_axis_name="s")

    @functools.partial(
        pl.kernel,
        mesh=mesh,
        out_type=jax.ShapeDtypeStruct((b, _D), jnp.float32),
        scratch_types=[
            pltpu.VMEM((_G + 24,), jnp.int32),
            pltpu.VMEM((nmeta,), jnp.int32),
            pltpu.VMEM((_G, _D), jnp.float32),
            pltpu.VMEM((_NBUF, _CHUNK, _D), jnp.float32),
            pltpu.SemaphoreType.DMA((_NBUF,)),
        ],
    )
    def k(table_hbm, starts_hbm, meta_hbm, out_hbm,
          starts_v, meta_v, table_v, bufs, osem):
        wid = lax.axis_index("s") * nc + lax.axis_index("c")
        base = jnp.minimum(wid * per_w, b - per_w)
        pltpu.sync_copy(starts_hbm, starts_v)
        pltpu.sync_copy(meta_hbm, meta_v)
        pltpu.sync_copy(table_hbm, table_v)

        def sread(ref, i):
            return ref[pl.ds(i, _L)][0]

        offs = [min(j * _CHUNK, per_w - _CHUNK) for j in range(nch)]

        oo = [None] * nch
        for j in range(nch):
            r0 = base + offs[j]
            m = sread(meta_v, wid * nch + j)
            fg = m & 0xFFFF
            nr = m >> 16
            bf = j % _NBUF
            if j >= _NBUF:
                oo[j - _NBUF].wait()
            bf_ref = bufs.at[bf]

            def body(i, _, r0=r0, fg=fg, bf_ref=bf_ref):
                g = fg + i
                sv = starts_v[pl.ds(g, _L)]
                ls = jnp.maximum(sv[0] - r0, 0)
                le = jnp.minimum(sv[1] - r0, _CHUNK)
                row = [table_v[g, pl.ds(c * _L, _L)] for c in range(_D // _L)]

                def rbody(r, _, row=row, bf_ref=bf_ref):
                    for c in range(_D // _L):
                        bf_ref[r, pl.ds(c * _L, _L)] = row[c]
                    return 0

                lax.fori_loop(ls, le, rbody, 0)
                return 0

            lax.fori_loop(0, nr, body, 0)
            oo[j] = pltpu.async_copy(
                bf_ref.at[pl.ds(0, _CHUNK)],
                out_hbm.at[pl.ds(r0, _CHUNK)],
                osem.at[bf],
            )
        for j in range(max(0, nch - _NBUF), nch):
            oo[j].wait()

    return k(table, starts, meta)


def kernel(graph_feat, batch):
    b = batch.shape[0]
    info = plsc.get_sparse_core_info()
    nw = info.num_cores * info.num_subcores
    per_w = -(-b // nw)
    per_w = -(-per_w // 8) * 8
    per_w = max(per_w, _CHUNK)
    nch = -(-per_w // _CHUNK)

    idx = batch.astype(jnp.int32)
    starts = jnp.searchsorted(idx, jnp.arange(_G + 24, dtype=jnp.int32))
    starts = jnp.minimum(starts, jnp.int32(b)).astype(jnp.int32)

    r0s = []
    for w in range(nw):
        bw = min(w * per_w, b - per_w)
        for j in range(nch):
            r0s.append(bw + min(j * _CHUNK, per_w - _CHUNK))
    r0s = jnp.asarray(r0s, dtype=jnp.int32)
    fg = jnp.searchsorted(starts[1:_G + 1], r0s, side="right")
    lg = jnp.searchsorted(starts[:_G], r0s + _CHUNK, side="left")
    meta = (fg | ((lg - fg) << 16)).astype(jnp.int32)
    nmeta = -(-meta.shape[0] // 8) * 8
    meta = jnp.pad(meta, (0, nmeta + 16 - meta.shape[0]))

    return _unpool(graph_feat, starts, meta, b=b, per_w=per_w, nch=nch)
